# baseline (speedup 1.0000x reference)
import functools

import jax
import jax.numpy as jnp
from jax import lax
from jax.experimental import pallas as pl
from jax.experimental.pallas import tpu as pltpu
from jax.experimental.pallas import tpu_sc as plsc

N = 10000
H = 128
A = 11
E = 320000

NC = 2
NS = 16
NW = NC * NS
CHUNK = 128
NCHUNK = 79
EW = E // NW
EPW = NCHUNK * CHUNK
DBITS = 14
PADV = N
ACC_ROWS = N + 16
OUT_STRIDE = 624
OUT_SPAN = 640

BM = 1000
NB = N // BM


def _transform_body(h_ref, w_ref, out_ref):
    out_ref[0] = jnp.dot(h_ref[...].astype(jnp.bfloat16), w_ref[0],
                         preferred_element_type=jnp.float32)


def _tc_transform(h):
    def call(w):
        return pl.pallas_call(
            _transform_body,
            grid=(NB, A),
            in_specs=[
                pl.BlockSpec((BM, H), lambda i, t: (i, 0)),
                pl.BlockSpec((1, H, H), lambda i, t: (t, 0, 0)),
            ],
            out_specs=pl.BlockSpec((1, BM, H), lambda i, t: (t, i, 0)),
            out_shape=jax.ShapeDtypeStruct((A, N, H), jnp.float32),
        )(h, w)
    return call


def _combine_transform_body(p_ref, b_ref, w_ref, out_ref):
    h = jax.nn.relu(p_ref[0] + p_ref[1] + b_ref[0])
    out_ref[0] = jnp.dot(h.astype(jnp.bfloat16), w_ref[0],
                         preferred_element_type=jnp.float32)


def _tc_combine_transform(p, b, w):
    return pl.pallas_call(
        _combine_transform_body,
        grid=(NB, A),
        in_specs=[
            pl.BlockSpec((2, BM, H), lambda i, t: (0, i, 0)),
            pl.BlockSpec((1, H), lambda i, t: (0, 0)),
            pl.BlockSpec((1, H, H), lambda i, t: (t, 0, 0)),
        ],
        out_specs=pl.BlockSpec((1, BM, H), lambda i, t: (t, i, 0)),
        out_shape=jax.ShapeDtypeStruct((A, N, H), jnp.float32),
    )(p, b, w)


def _final_body(p_ref, b_ref, wa_ref, ba_ref, out_ref):
    h = jax.nn.relu(p_ref[0] + p_ref[1] + b_ref[0])
    out_ref[...] = jnp.dot(h.astype(jnp.bfloat16), wa_ref[...],
                           preferred_element_type=jnp.float32) + ba_ref[0]


def _tc_final(p, b, w_add, b_add):
    return pl.pallas_call(
        _final_body,
        grid=(NB,),
        in_specs=[
            pl.BlockSpec((2, BM, H), lambda i: (0, i, 0)),
            pl.BlockSpec((1, H), lambda i: (0, 0)),
            pl.BlockSpec((H, H), lambda i: (0, 0)),
            pl.BlockSpec((1, H), lambda i: (0, 0)),
        ],
        out_specs=pl.BlockSpec((BM, H), lambda i: (i, 0)),
        out_shape=jax.ShapeDtypeStruct((N, H), jnp.float32),
    )(p, b, w_add, b_add)


NBUF = 2
NOUTER = NCHUNK // NBUF
NTAIL = NCHUNK - NBUF * NOUTER


def _sc_body(table, pidx, zeros, out,
             pv, gslot, dslot, r0, r1, acc, gsem, ssem):
    rows = (r0, r1)
    c = lax.axis_index("c")
    s = lax.axis_index("s")
    wid = s * NC + c

    pltpu.sync_copy(zeros, acc.at[pl.ds(s * OUT_STRIDE, OUT_SPAN)])
    pltpu.sync_copy(pidx.at[wid], pv)
    plsc.subcore_barrier()

    def unpack(j, b):
        for k in range(CHUNK // 16):
            sl = pl.ds(k * 16, 16)
            p = pv[j, sl]
            gslot[b, sl] = jax.lax.shift_right_logical(p, DBITS)
            dslot[b, sl] = jax.lax.bitwise_and(p, (1 << DBITS) - 1)

    for b in range(NBUF):
        unpack(b, b)
        pltpu.async_copy(table.at[gslot.at[b]], rows[b], gsem[b])

    def step(i, carry):
        j0 = i * NBUF
        for b in range(NBUF):
            pltpu.make_async_copy(table.at[gslot.at[b]], rows[b],
                                  gsem[b]).wait()
            pltpu.async_copy(rows[b], acc.at[dslot.at[b]], ssem[b],
                             add=True)
        for b in range(NBUF):
            j2 = j0 + b + NBUF

            @pl.when(j2 < NCHUNK)
            def _():
                pltpu.make_async_copy(rows[b], acc.at[dslot.at[b]],
                                      ssem[b]).wait()
                unpack(j2, b)
                pltpu.async_copy(table.at[gslot.at[b]], rows[b], gsem[b])
        return carry

    lax.fori_loop(0, NOUTER, step, 0)
    for t in range(NTAIL):
        pltpu.make_async_copy(table.at[gslot.at[t]], rows[t], gsem[t]).wait()
        pltpu.async_copy(rows[t], acc.at[dslot.at[t]], ssem[t], add=True)
    for b in range(NBUF):
        pltpu.make_async_copy(rows[b], acc.at[dslot.at[b]], ssem[b]).wait()
    plsc.subcore_barrier()

    pltpu.sync_copy(acc.at[pl.ds(s * OUT_STRIDE, OUT_SPAN)],
                    out.at[c, pl.ds(s * OUT_STRIDE, OUT_SPAN)])


_SC_MESH = plsc.VectorSubcoreMesh(core_axis_name="c", subcore_axis_name="s")

_sc_aggregate = pl.kernel(
    _sc_body,
    out_type=jax.ShapeDtypeStruct((NC, N, H), jnp.float32),
    mesh=_SC_MESH,
    scratch_types=[
        pltpu.VMEM((NCHUNK, CHUNK), jnp.int32),
        pltpu.VMEM((NBUF, CHUNK), jnp.int32),
        pltpu.VMEM((NBUF, CHUNK), jnp.int32),
        pltpu.VMEM((CHUNK, H), jnp.float32),
        pltpu.VMEM((CHUNK, H), jnp.float32),
        pltpu.VMEM_SHARED((ACC_ROWS, H), jnp.float32),
        [pltpu.SemaphoreType.DMA] * NBUF,
        [pltpu.SemaphoreType.DMA] * NBUF,
    ],
)


def kernel(x, edge_index, edge_type, W1, b1, W2, b2, W_add, b_add):
    src = edge_index[0]
    dst = edge_index[1]

    packed = (((edge_type * N + src) << DBITS) | dst).reshape(NW, EW)
    pidx = jnp.pad(packed, ((0, 0), (0, EPW - EW)),
                   constant_values=PADV).reshape(NW, NCHUNK, CHUNK)
    zeros = jnp.zeros((OUT_SPAN, H), jnp.float32)
    b1r = b1.reshape(1, H)
    b2r = b2.reshape(1, H)
    bar = b_add.reshape(1, H)

    w1h = W1.astype(jnp.bfloat16)
    w2h = W2.astype(jnp.bfloat16)
    wah = W_add.astype(jnp.bfloat16)

    table1 = _tc_transform(x)(w1h).reshape(A * N, H)
    p1 = _sc_aggregate(table1, pidx, zeros)
    table2 = _tc_combine_transform(p1, b1r, w2h).reshape(A * N, H)
    p2 = _sc_aggregate(table2, pidx, zeros)
    return _tc_final(p2, b2r, wah, bar)


# speedup vs baseline: 1.6724x; 1.6724x over previous
import functools

import jax
import jax.numpy as jnp
from jax import lax
from jax.experimental import pallas as pl
from jax.experimental.pallas import tpu as pltpu
from jax.experimental.pallas import tpu_sc as plsc

N = 10000
H = 128
A = 11
E = 320000

NC = 2
NS = 16
NW = NC * NS
CHUNK = 80
NCHUNK = 125
EW = CHUNK * NCHUNK
DBITS = 14
ACC_ROWS = N
OUT_STRIDE = 624
OUT_SPAN = 640

BM = 1000
NB = N // BM


def _transform_body(h_ref, w_ref, out_ref):
    out_ref[0] = jnp.dot(h_ref[...].astype(jnp.bfloat16), w_ref[0],
                         preferred_element_type=jnp.float32)


def _tc_transform(h):
    def call(w):
        return pl.pallas_call(
            _transform_body,
            grid=(NB, A),
            in_specs=[
                pl.BlockSpec((BM, H), lambda i, t: (i, 0)),
                pl.BlockSpec((1, H, H), lambda i, t: (t, 0, 0)),
            ],
            out_specs=pl.BlockSpec((1, BM, H), lambda i, t: (t, i, 0)),
            out_shape=jax.ShapeDtypeStruct((A, N, H), jnp.float32),
        )(h, w)
    return call


def _combine_transform_body(p_ref, b_ref, w_ref, out_ref):
    h = jax.nn.relu(p_ref[0] + p_ref[1] + b_ref[0])
    out_ref[0] = jnp.dot(h.astype(jnp.bfloat16), w_ref[0],
                         preferred_element_type=jnp.float32)


def _tc_combine_transform(p, b, w):
    return pl.pallas_call(
        _combine_transform_body,
        grid=(NB, A),
        in_specs=[
            pl.BlockSpec((2, BM, H), lambda i, t: (0, i, 0)),
            pl.BlockSpec((1, H), lambda i, t: (0, 0)),
            pl.BlockSpec((1, H, H), lambda i, t: (t, 0, 0)),
        ],
        out_specs=pl.BlockSpec((1, BM, H), lambda i, t: (t, i, 0)),
        out_shape=jax.ShapeDtypeStruct((A, N, H), jnp.float32),
    )(p, b, w)


def _final_body(p_ref, b_ref, wa_ref, ba_ref, out_ref):
    h = jax.nn.relu(p_ref[0] + p_ref[1] + b_ref[0])
    out_ref[...] = jnp.dot(h.astype(jnp.bfloat16), wa_ref[...],
                           preferred_element_type=jnp.float32) + ba_ref[0]


def _tc_final(p, b, w_add, b_add):
    return pl.pallas_call(
        _final_body,
        grid=(NB,),
        in_specs=[
            pl.BlockSpec((2, BM, H), lambda i: (0, i, 0)),
            pl.BlockSpec((1, H), lambda i: (0, 0)),
            pl.BlockSpec((H, H), lambda i: (0, 0)),
            pl.BlockSpec((1, H), lambda i: (0, 0)),
        ],
        out_specs=pl.BlockSpec((BM, H), lambda i: (i, 0)),
        out_shape=jax.ShapeDtypeStruct((N, H), jnp.float32),
    )(p, b, w_add, b_add)


NBUF = 3
NOUTER = NCHUNK // NBUF
NTAIL = NCHUNK - NBUF * NOUTER


def _sc_body(table, pidx, zeros, out,
             pv, gslot, dslot, r0, r1, r2, acc, gsem, ssem):
    rows = (r0, r1, r2)
    c = lax.axis_index("c")
    s = lax.axis_index("s")
    wid = s * NC + c

    pltpu.sync_copy(zeros, acc.at[pl.ds(s * OUT_STRIDE, OUT_SPAN)])
    pltpu.sync_copy(pidx.at[wid], pv)
    plsc.subcore_barrier()

    def unpack(j, b):
        for k in range(CHUNK // 16):
            sl = pl.ds(k * 16, 16)
            p = pv[j, sl]
            gslot[b, sl] = jax.lax.shift_right_logical(p, DBITS)
            dslot[b, sl] = jax.lax.bitwise_and(p, (1 << DBITS) - 1)

    for b in range(NBUF):
        unpack(b, b)
        pltpu.async_copy(table.at[gslot.at[b]], rows[b], gsem[b])

    def step(i, carry):
        j0 = i * NBUF
        for b in range(NBUF):
            pltpu.make_async_copy(table.at[gslot.at[b]], rows[b],
                                  gsem[b]).wait()
            pltpu.async_copy(rows[b], acc.at[dslot.at[b]], ssem[b],
                             add=True)
        for b in range(NBUF):
            j2 = j0 + b + NBUF

            @pl.when(j2 < NCHUNK)
            def _():
                pltpu.make_async_copy(rows[b], acc.at[dslot.at[b]],
                                      ssem[b]).wait()
                unpack(j2, b)
                pltpu.async_copy(table.at[gslot.at[b]], rows[b], gsem[b])
        return carry

    lax.fori_loop(0, NOUTER, step, 0)
    for t in range(NTAIL):
        pltpu.make_async_copy(table.at[gslot.at[t]], rows[t], gsem[t]).wait()
        pltpu.async_copy(rows[t], acc.at[dslot.at[t]], ssem[t], add=True)
    for b in range(NBUF):
        pltpu.make_async_copy(rows[b], acc.at[dslot.at[b]], ssem[b]).wait()
    plsc.subcore_barrier()

    pltpu.sync_copy(acc.at[pl.ds(s * OUT_STRIDE, OUT_SPAN)],
                    out.at[c, pl.ds(s * OUT_STRIDE, OUT_SPAN)])


_SC_MESH = plsc.VectorSubcoreMesh(core_axis_name="c", subcore_axis_name="s")

_sc_aggregate = pl.kernel(
    _sc_body,
    out_type=jax.ShapeDtypeStruct((NC, N, H), jnp.float32),
    mesh=_SC_MESH,
    scratch_types=[
        pltpu.VMEM((NCHUNK, CHUNK), jnp.int32),
        pltpu.VMEM((NBUF, CHUNK), jnp.int32),
        pltpu.VMEM((NBUF, CHUNK), jnp.int32),
        pltpu.VMEM((CHUNK, H), jnp.float32),
        pltpu.VMEM((CHUNK, H), jnp.float32),
        pltpu.VMEM((CHUNK, H), jnp.float32),
        pltpu.VMEM_SHARED((ACC_ROWS, H), jnp.float32),
        [pltpu.SemaphoreType.DMA] * NBUF,
        [pltpu.SemaphoreType.DMA] * NBUF,
    ],
)


def kernel(x, edge_index, edge_type, W1, b1, W2, b2, W_add, b_add):
    src = edge_index[0]
    dst = edge_index[1]

    pidx = (((edge_type * N + src) << DBITS) | dst).reshape(NW, NCHUNK, CHUNK)
    zeros = jnp.zeros((OUT_SPAN, H), jnp.float32)
    b1r = b1.reshape(1, H)
    b2r = b2.reshape(1, H)
    bar = b_add.reshape(1, H)

    w1h = W1.astype(jnp.bfloat16)
    w2h = W2.astype(jnp.bfloat16)
    wah = W_add.astype(jnp.bfloat16)

    table1 = _tc_transform(x)(w1h).reshape(A * N, H)
    p1 = _sc_aggregate(table1, pidx, zeros)
    table2 = _tc_combine_transform(p1, b1r, w2h).reshape(A * N, H)
    p2 = _sc_aggregate(table2, pidx, zeros)
    return _tc_final(p2, b2r, wah, bar)


# speedup vs baseline: 1.6806x; 1.0049x over previous
import functools

import jax
import jax.numpy as jnp
from jax import lax
from jax.experimental import pallas as pl
from jax.experimental.pallas import tpu as pltpu
from jax.experimental.pallas import tpu_sc as plsc

N = 10000
H = 128
A = 11
E = 320000

NC = 2
NS = 16
NW = NC * NS
CHUNK = 80
NCHUNK = 125
EW = CHUNK * NCHUNK
DBITS = 14
ACC_ROWS = N
OUT_STRIDE = 624
OUT_SPAN = 640

BM = 1000
NB = N // BM


def _transform_body(h_ref, w_ref, out_ref):
    out_ref[0] = jnp.dot(h_ref[...].astype(jnp.bfloat16), w_ref[0],
                         preferred_element_type=jnp.float32)


def _tc_transform(h):
    def call(w):
        return pl.pallas_call(
            _transform_body,
            grid=(NB, A),
            in_specs=[
                pl.BlockSpec((BM, H), lambda i, t: (i, 0)),
                pl.BlockSpec((1, H, H), lambda i, t: (t, 0, 0)),
            ],
            out_specs=pl.BlockSpec((1, BM, H), lambda i, t: (t, i, 0)),
            out_shape=jax.ShapeDtypeStruct((A, N, H), jnp.float32),
        )(h, w)
    return call


def _combine_transform_body(p_ref, b_ref, w_ref, out_ref):
    h = jax.nn.relu(p_ref[0] + p_ref[1] + b_ref[0])
    out_ref[0] = jnp.dot(h.astype(jnp.bfloat16), w_ref[0],
                         preferred_element_type=jnp.float32)


def _tc_combine_transform(p, b, w):
    return pl.pallas_call(
        _combine_transform_body,
        grid=(NB, A),
        in_specs=[
            pl.BlockSpec((2, BM, H), lambda i, t: (0, i, 0)),
            pl.BlockSpec((1, H), lambda i, t: (0, 0)),
            pl.BlockSpec((1, H, H), lambda i, t: (t, 0, 0)),
        ],
        out_specs=pl.BlockSpec((1, BM, H), lambda i, t: (t, i, 0)),
        out_shape=jax.ShapeDtypeStruct((A, N, H), jnp.float32),
    )(p, b, w)


def _final_body(p_ref, b_ref, wa_ref, ba_ref, out_ref):
    h = jax.nn.relu(p_ref[0] + p_ref[1] + b_ref[0])
    out_ref[...] = jnp.dot(h.astype(jnp.bfloat16), wa_ref[...],
                           preferred_element_type=jnp.float32) + ba_ref[0]


def _tc_final(p, b, w_add, b_add):
    return pl.pallas_call(
        _final_body,
        grid=(NB,),
        in_specs=[
            pl.BlockSpec((2, BM, H), lambda i: (0, i, 0)),
            pl.BlockSpec((1, H), lambda i: (0, 0)),
            pl.BlockSpec((H, H), lambda i: (0, 0)),
            pl.BlockSpec((1, H), lambda i: (0, 0)),
        ],
        out_specs=pl.BlockSpec((BM, H), lambda i: (i, 0)),
        out_shape=jax.ShapeDtypeStruct((N, H), jnp.float32),
    )(p, b, w_add, b_add)


NBUF = 3
NOUTER = NCHUNK // NBUF
NTAIL = NCHUNK - NBUF * NOUTER


def _sc_body(table, pidx, zeros, out,
             pv, gslot, dslot, r0, r1, r2, acc, gsem, ssem):
    rows = (r0, r1, r2)
    c = lax.axis_index("c")
    s = lax.axis_index("s")
    wid = s * NC + c

    pltpu.sync_copy(zeros, acc.at[pl.ds(s * OUT_STRIDE, OUT_SPAN)])
    pltpu.sync_copy(pidx.at[wid], pv)
    plsc.subcore_barrier()

    def unpack(j, b):
        for k in range(CHUNK // 16):
            sl = pl.ds(k * 16, 16)
            p = pv[j, sl]
            gslot[b, sl] = jax.lax.shift_right_logical(p, DBITS)
            dslot[b, sl] = jax.lax.bitwise_and(p, (1 << DBITS) - 1)

    for b in range(NBUF):
        unpack(b, b)
        pltpu.async_copy(table.at[gslot.at[b]], rows[b], gsem[b])

    def step(i, carry):
        bank = lax.bitwise_and(i, 1) * NBUF
        nxt = lax.bitwise_and(i + 1, 1) * NBUF
        for b in range(NBUF):
            pltpu.make_async_copy(table.at[gslot.at[bank + b]], rows[b],
                                  gsem[b]).wait()
            pltpu.async_copy(rows[b], acc.at[dslot.at[bank + b]], ssem[b],
                             add=True)
        for b in range(NBUF):
            j2 = i * NBUF + b + NBUF

            @pl.when(j2 < NCHUNK)
            def _():
                unpack(j2, nxt + b)
                pltpu.make_async_copy(rows[b], acc.at[dslot.at[bank + b]],
                                      ssem[b]).wait()
                pltpu.async_copy(table.at[gslot.at[nxt + b]], rows[b],
                                 gsem[b])
        return carry

    lax.fori_loop(0, NOUTER, step, 0)
    tb = (NOUTER % 2) * NBUF
    for t in range(NTAIL):
        pltpu.make_async_copy(table.at[gslot.at[tb + t]], rows[t],
                              gsem[t]).wait()
        pltpu.async_copy(rows[t], acc.at[dslot.at[tb + t]], ssem[t],
                         add=True)
    for b in range(NBUF):
        pltpu.make_async_copy(rows[b], acc.at[dslot.at[b]], ssem[b]).wait()
    plsc.subcore_barrier()

    pltpu.sync_copy(acc.at[pl.ds(s * OUT_STRIDE, OUT_SPAN)],
                    out.at[c, pl.ds(s * OUT_STRIDE, OUT_SPAN)])


_SC_MESH = plsc.VectorSubcoreMesh(core_axis_name="c", subcore_axis_name="s")

_sc_aggregate = pl.kernel(
    _sc_body,
    out_type=jax.ShapeDtypeStruct((NC, N, H), jnp.float32),
    mesh=_SC_MESH,
    scratch_types=[
        pltpu.VMEM((NCHUNK, CHUNK), jnp.int32),
        pltpu.VMEM((2 * NBUF, CHUNK), jnp.int32),
        pltpu.VMEM((2 * NBUF, CHUNK), jnp.int32),
        pltpu.VMEM((CHUNK, H), jnp.float32),
        pltpu.VMEM((CHUNK, H), jnp.float32),
        pltpu.VMEM((CHUNK, H), jnp.float32),
        pltpu.VMEM_SHARED((ACC_ROWS, H), jnp.float32),
        [pltpu.SemaphoreType.DMA] * NBUF,
        [pltpu.SemaphoreType.DMA] * NBUF,
    ],
)


def kernel(x, edge_index, edge_type, W1, b1, W2, b2, W_add, b_add):
    src = edge_index[0]
    dst = edge_index[1]

    pidx = (((edge_type * N + src) << DBITS) | dst).reshape(NW, NCHUNK, CHUNK)
    zeros = jnp.zeros((OUT_SPAN, H), jnp.float32)
    b1r = b1.reshape(1, H)
    b2r = b2.reshape(1, H)
    bar = b_add.reshape(1, H)

    w1h = W1.astype(jnp.bfloat16)
    w2h = W2.astype(jnp.bfloat16)
    wah = W_add.astype(jnp.bfloat16)

    table1 = _tc_transform(x)(w1h).reshape(A * N, H)
    p1 = _sc_aggregate(table1, pidx, zeros)
    table2 = _tc_combine_transform(p1, b1r, w2h).reshape(A * N, H)
    p2 = _sc_aggregate(table2, pidx, zeros)
    return _tc_final(p2, b2r, wah, bar)


# speedup vs baseline: 1.6817x; 1.0006x over previous
import jax
import jax.numpy as jnp
from jax import lax
from jax.experimental import pallas as pl
from jax.experimental.pallas import tpu as pltpu
from jax.experimental.pallas import tpu_sc as plsc

N = 10000
H = 128
A = 11
E = 320000

NC = 2
NS = 16
NW = NC * NS
CHUNK = 80
NCHUNK = 125
EW = CHUNK * NCHUNK
DBITS = 14
ACC_ROWS = N
OUT_STRIDE = 624
OUT_SPAN = 640

BM = 1000
NB = N // BM


def _transform_body(h_ref, w_ref, out_ref):
    out_ref[0] = jnp.dot(h_ref[...].astype(jnp.bfloat16), w_ref[0],
                         preferred_element_type=jnp.float32)


def _tc_transform(h):
    def call(w):
        return pl.pallas_call(
            _transform_body,
            grid=(NB, A),
            in_specs=[
                pl.BlockSpec((BM, H), lambda i, t: (i, 0)),
                pl.BlockSpec((1, H, H), lambda i, t: (t, 0, 0)),
            ],
            out_specs=pl.BlockSpec((1, BM, H), lambda i, t: (t, i, 0)),
            out_shape=jax.ShapeDtypeStruct((A, N, H), jnp.float32),
        )(h, w)
    return call


def _combine_transform_body(p_ref, b_ref, w_ref, out_ref):
    h = jax.nn.relu(p_ref[0] + p_ref[1] + b_ref[0])
    out_ref[0] = jnp.dot(h.astype(jnp.bfloat16), w_ref[0],
                         preferred_element_type=jnp.float32)


def _tc_combine_transform(p, b, w):
    return pl.pallas_call(
        _combine_transform_body,
        grid=(NB, A),
        in_specs=[
            pl.BlockSpec((2, BM, H), lambda i, t: (0, i, 0)),
            pl.BlockSpec((1, H), lambda i, t: (0, 0)),
            pl.BlockSpec((1, H, H), lambda i, t: (t, 0, 0)),
        ],
        out_specs=pl.BlockSpec((1, BM, H), lambda i, t: (t, i, 0)),
        out_shape=jax.ShapeDtypeStruct((A, N, H), jnp.float32),
    )(p, b, w)


def _final_body(p_ref, b_ref, wa_ref, ba_ref, out_ref):
    h = jax.nn.relu(p_ref[0] + p_ref[1] + b_ref[0])
    out_ref[...] = jnp.dot(h.astype(jnp.bfloat16), wa_ref[...],
                           preferred_element_type=jnp.float32) + ba_ref[0]


def _tc_final(p, b, w_add, b_add):
    return pl.pallas_call(
        _final_body,
        grid=(NB,),
        in_specs=[
            pl.BlockSpec((2, BM, H), lambda i: (0, i, 0)),
            pl.BlockSpec((1, H), lambda i: (0, 0)),
            pl.BlockSpec((H, H), lambda i: (0, 0)),
            pl.BlockSpec((1, H), lambda i: (0, 0)),
        ],
        out_specs=pl.BlockSpec((BM, H), lambda i: (i, 0)),
        out_shape=jax.ShapeDtypeStruct((N, H), jnp.float32),
    )(p, b, w_add, b_add)


NBUF = 3
NOUTER = NCHUNK // NBUF
NTAIL = NCHUNK - NBUF * NOUTER


def _sc_body(table, pidx, zeros, out,
             pv, gslot, dslot, r0, r1, r2, acc, gsem, ssem):
    rows = (r0, r1, r2)
    c = lax.axis_index("c")
    s = lax.axis_index("s")
    wid = s * NC + c

    pltpu.sync_copy(zeros, acc.at[pl.ds(s * OUT_STRIDE, OUT_SPAN)])
    pltpu.sync_copy(pidx.at[wid], pv)
    plsc.subcore_barrier()

    def unpack(j, b):
        for k in range(CHUNK // 16):
            sl = pl.ds(k * 16, 16)
            p = pv[j, sl]
            gslot[b, sl] = jax.lax.shift_right_logical(p, DBITS)
            dslot[b, sl] = jax.lax.bitwise_and(p, (1 << DBITS) - 1)

    for b in range(NBUF):
        unpack(b, b)
        pltpu.async_copy(table.at[gslot.at[b]], rows[b], gsem[b])

    def step(i, carry):
        bank = lax.bitwise_and(i, 1) * NBUF
        nxt = lax.bitwise_and(i + 1, 1) * NBUF
        for b in range(NBUF):
            pltpu.make_async_copy(table.at[gslot.at[bank + b]], rows[b],
                                  gsem[b]).wait()
            pltpu.async_copy(rows[b], acc.at[dslot.at[bank + b]], ssem[b],
                             add=True)
        for b in range(NBUF):
            j2 = i * NBUF + b + NBUF

            @pl.when(j2 < NCHUNK)
            def _():
                unpack(j2, nxt + b)
                pltpu.make_async_copy(rows[b], acc.at[dslot.at[bank + b]],
                                      ssem[b]).wait()
                pltpu.async_copy(table.at[gslot.at[nxt + b]], rows[b],
                                 gsem[b])
        return carry

    lax.fori_loop(0, NOUTER, step, 0)
    tb = (NOUTER % 2) * NBUF
    for t in range(NTAIL):
        pltpu.make_async_copy(table.at[gslot.at[tb + t]], rows[t],
                              gsem[t]).wait()
        pltpu.async_copy(rows[t], acc.at[dslot.at[tb + t]], ssem[t],
                         add=True)
    for b in range(NBUF):
        pltpu.make_async_copy(rows[b], acc.at[dslot.at[b]], ssem[b]).wait()
    plsc.subcore_barrier()

    pltpu.sync_copy(acc.at[pl.ds(s * OUT_STRIDE, OUT_SPAN)],
                    out.at[c, pl.ds(s * OUT_STRIDE, OUT_SPAN)])


_SC_MESH = plsc.VectorSubcoreMesh(core_axis_name="c", subcore_axis_name="s")

_sc_aggregate = pl.kernel(
    _sc_body,
    out_type=jax.ShapeDtypeStruct((NC, N, H), jnp.float32),
    mesh=_SC_MESH,
    scratch_types=[
        pltpu.VMEM((NCHUNK, CHUNK), jnp.int32),
        pltpu.VMEM((2 * NBUF, CHUNK), jnp.int32),
        pltpu.VMEM((2 * NBUF, CHUNK), jnp.int32),
        pltpu.VMEM((CHUNK, H), jnp.float32),
        pltpu.VMEM((CHUNK, H), jnp.float32),
        pltpu.VMEM((CHUNK, H), jnp.float32),
        pltpu.VMEM_SHARED((ACC_ROWS, H), jnp.float32),
        [pltpu.SemaphoreType.DMA] * NBUF,
        [pltpu.SemaphoreType.DMA] * NBUF,
    ],
)


def kernel(x, edge_index, edge_type, W1, b1, W2, b2, W_add, b_add):
    src = edge_index[0]
    dst = edge_index[1]

    pidx = (((edge_type * N + src) << DBITS) | dst).reshape(NW, NCHUNK, CHUNK)
    zeros = jnp.zeros((OUT_SPAN, H), jnp.float32)
    b1r = b1.reshape(1, H)
    b2r = b2.reshape(1, H)
    bar = b_add.reshape(1, H)

    w1h = W1.astype(jnp.bfloat16)
    w2h = W2.astype(jnp.bfloat16)
    wah = W_add.astype(jnp.bfloat16)

    table1 = _tc_transform(x)(w1h).reshape(A * N, H)
    p1 = _sc_aggregate(table1, pidx, zeros)
    table2 = _tc_combine_transform(p1, b1r, w2h).reshape(A * N, H)
    p2 = _sc_aggregate(table2, pidx, zeros)
    return _tc_final(p2, b2r, wah, bar)
